# baseline (speedup 1.0000x reference)
import functools
import jax
import jax.numpy as jnp
from jax import lax
from jax.experimental import pallas as pl
from jax.experimental.pallas import tpu as pltpu
from jax.experimental.pallas import tpu_sc as plsc

B = 32
L = 4
V = 100000
N = B * L
PLACEHOLDER = -1

LANES = 16
CHUNK = 4000
NCHUNK = V // CHUNK
ITERS = CHUNK // LANES
I32_MAX = 2147483647


def _sc_body(ids_hbm, d_hbm, t_hbm, bonus_hbm, u_hbm, q_hbm, out_hbm,
             ids_v, u_v, bonus_v, orow_v,
             t0, t1, t2, t3, d0, d1, d2, d3, qb):
    w = lax.axis_index("s") * 2 + lax.axis_index("c")
    iota = lax.iota(jnp.int32, LANES)

    pltpu.sync_copy(ids_hbm, ids_v)
    pltpu.sync_copy(u_hbm, u_v)
    pltpu.sync_copy(bonus_hbm, bonus_v)

    row_sel = jnp.minimum(4 * w + iota, jnp.int32(N - 1))
    idv = plsc.load_gather(ids_v, [row_sel])
    uv = plsc.load_gather(u_v, [row_sel])

    tbufs = (t0, t1, t2, t3)
    dbufs = (d0, d1, d2, d3)

    zero_f = jnp.zeros((LANES,), jnp.float32)
    neg1_f = jnp.full((LANES,), -1.0, jnp.float32)
    zero_i = jnp.zeros((LANES,), jnp.int32)

    m = [neg1_f] * 4
    mi = [zero_i] * 4
    dp_keep = zero_f
    tp_keep = zero_f

    for c in range(NCHUNK):
        c0 = c * CHUNK
        for i in range(4):
            off = (4 * w + i) * V + c0
            pltpu.sync_copy(t_hbm.at[pl.ds(off, CHUNK)], tbufs[i])
            pltpu.sync_copy(d_hbm.at[pl.ds(off, CHUNK)], dbufs[i])
        pltpu.sync_copy(q_hbm.at[pl.ds(w * V + c0, CHUNK)], qb)

        def inner(j, carry):
            m0, m1, m2, m3, i0, i1, i2, i3 = carry
            sl = pl.ds(j * LANES, LANES)
            qv = qb[sl]
            vidx = (jnp.int32(c0) + j * LANES) + iota
            ms = [m0, m1, m2, m3]
            ix = [i0, i1, i2, i3]
            for i in range(4):
                p = jnp.maximum(tbufs[i][sl] - dbufs[i][sl], 0.0)
                r = p / qv
                better = r > ms[i]
                ms[i] = jnp.maximum(ms[i], r)
                ix[i] = jnp.where(better, vidx, ix[i])
            return ms[0], ms[1], ms[2], ms[3], ix[0], ix[1], ix[2], ix[3]

        m[0], m[1], m[2], m[3], mi[0], mi[1], mi[2], mi[3] = lax.fori_loop(
            0, ITERS, inner,
            (m[0], m[1], m[2], m[3], mi[0], mi[1], mi[2], mi[3]),
            unroll=2)

        in_r = (idv >= c0) & (idv < c0 + CHUNK)
        pos = jnp.clip(idv - c0, 0, CHUNK - 1)
        for i in range(4):
            keep = in_r & (iota == i)
            g_t = plsc.load_gather(tbufs[i], [pos])
            g_d = plsc.load_gather(dbufs[i], [pos])
            tp_keep = jnp.where(keep, g_t, tp_keep)
            dp_keep = jnp.where(keep, g_d, dp_keep)

    rec = zero_i
    for i in range(4):
        mv = jnp.max(m[i])
        cand = jnp.where(m[i] == mv, mi[i], I32_MAX)
        best = jnp.min(cand)
        rec = jnp.where(iota == i, best, rec)

    safe_dp = jnp.where(dp_keep > 0, dp_keep, 1.0)
    acc = (dp_keep > 0) & (tp_keep / safe_dp >= uv)
    tok = jnp.where(acc, idv, rec)
    rej = jnp.where(acc | (iota >= 4), 0, 1).astype(jnp.int32)
    cs = plsc.cumsum(rej)
    excl = cs - rej
    write = (excl == 0) & (iota < 4)
    o = jnp.where(write, tok, jnp.int32(PLACEHOLDER))
    tot = jnp.max(jnp.where(iota < 4, cs, 0))
    bonusv = plsc.load_gather(bonus_v, [jnp.full((LANES,), w, jnp.int32)])
    o = jnp.where(iota == 4,
                  jnp.where(tot == 0, bonusv, jnp.int32(PLACEHOLDER)), o)
    orow_v[...] = o
    pltpu.sync_copy(orow_v, out_hbm.at[pl.ds(w * LANES, LANES)])


@jax.jit
def _sc_call(ids, dflat, tflat, bonus, u, qflat):
    mesh = plsc.VectorSubcoreMesh(core_axis_name="c", subcore_axis_name="s")
    f = pl.kernel(
        _sc_body,
        out_type=jax.ShapeDtypeStruct((B * LANES,), jnp.int32),
        mesh=mesh,
        scratch_types=[
            pltpu.VMEM((N,), jnp.int32),
            pltpu.VMEM((N,), jnp.float32),
            pltpu.VMEM((B,), jnp.int32),
            pltpu.VMEM((LANES,), jnp.int32),
        ] + [pltpu.VMEM((CHUNK,), jnp.float32)] * 9,
        compiler_params=pltpu.CompilerParams(needs_layout_passes=False),
    )
    return f(ids, dflat, tflat, bonus, u, qflat)


def kernel(draft_token_ids, cu_num_draft_tokens, draft_probs, target_probs,
           bonus_token_ids, uniform_probs, q):
    del cu_num_draft_tokens
    out = _sc_call(
        draft_token_ids,
        draft_probs.reshape(-1),
        target_probs.reshape(-1),
        bonus_token_ids,
        uniform_probs,
        q.reshape(-1),
    )
    return out.reshape(B, LANES)[:, :L + 1]


# speedup vs baseline: 1.5085x; 1.5085x over previous
import functools
import jax
import jax.numpy as jnp
from jax import lax
from jax.experimental import pallas as pl
from jax.experimental.pallas import tpu as pltpu
from jax.experimental.pallas import tpu_sc as plsc

B = 32
L = 4
V = 100000
N = B * L
PLACEHOLDER = -1

LANES = 16
CHUNK = 4000
NCHUNK = V // CHUNK
ITERS = CHUNK // LANES
I32_MAX = 2147483647


def _sc_body(ids_hbm, d_hbm, t_hbm, bonus_hbm, u_hbm, q_hbm, out_hbm,
             ids_v, u_v, bonus_v, orow_v,
             t0a, t1a, t2a, t3a, d0a, d1a, d2a, d3a, qa,
             t0b, t1b, t2b, t3b, d0b, d1b, d2b, d3b, qb,
             sema, semb):
    w = lax.axis_index("s") * 2 + lax.axis_index("c")
    iota = lax.iota(jnp.int32, LANES)

    tsets = ((t0a, t1a, t2a, t3a), (t0b, t1b, t2b, t3b))
    dsets = ((d0a, d1a, d2a, d3a), (d0b, d1b, d2b, d3b))
    qsets = (qa, qb)
    sems = (sema, semb)

    def issue(c, s):
        c0 = c * CHUNK
        cps = []
        for i in range(4):
            off = (4 * w + i) * V + c0
            cps.append(pltpu.async_copy(
                t_hbm.at[pl.ds(off, CHUNK)], tsets[s][i], sems[s]))
            cps.append(pltpu.async_copy(
                d_hbm.at[pl.ds(off, CHUNK)], dsets[s][i], sems[s]))
        cps.append(pltpu.async_copy(
            q_hbm.at[pl.ds(w * V + c0, CHUNK)], qsets[s], sems[s]))
        return cps

    pltpu.sync_copy(ids_hbm, ids_v)
    pltpu.sync_copy(u_hbm, u_v)
    pltpu.sync_copy(bonus_hbm, bonus_v)

    row_sel = jnp.minimum(4 * w + iota, jnp.int32(N - 1))
    idv = plsc.load_gather(ids_v, [row_sel])
    uv = plsc.load_gather(u_v, [row_sel])

    zero_f = jnp.zeros((LANES,), jnp.float32)
    neg1_f = jnp.full((LANES,), -1.0, jnp.float32)
    zero_i = jnp.zeros((LANES,), jnp.int32)

    m = [neg1_f] * 4
    mi = [zero_i] * 4
    dp_keep = zero_f
    tp_keep = zero_f

    cur = issue(0, 0)
    for c in range(NCHUNK):
        s = c % 2
        c0 = c * CHUNK
        for cp in cur:
            cp.wait()
        if c + 1 < NCHUNK:
            cur = issue(c + 1, (c + 1) % 2)
        tbufs = tsets[s]
        dbufs = dsets[s]
        qbuf = qsets[s]

        def inner(j, carry):
            m0, m1, m2, m3, i0, i1, i2, i3 = carry
            sl = pl.ds(j * LANES, LANES)
            rq = 1.0 / qbuf[sl]
            vidx = (jnp.int32(c0) + j * LANES) + iota
            ms = [m0, m1, m2, m3]
            ix = [i0, i1, i2, i3]
            for i in range(4):
                p = jnp.maximum(tbufs[i][sl] - dbufs[i][sl], 0.0)
                r = p * rq
                better = r > ms[i]
                ms[i] = jnp.maximum(ms[i], r)
                ix[i] = jnp.where(better, vidx, ix[i])
            return ms[0], ms[1], ms[2], ms[3], ix[0], ix[1], ix[2], ix[3]

        m[0], m[1], m[2], m[3], mi[0], mi[1], mi[2], mi[3] = lax.fori_loop(
            0, ITERS, inner,
            (m[0], m[1], m[2], m[3], mi[0], mi[1], mi[2], mi[3]),
            unroll=4)

        in_r = (idv >= c0) & (idv < c0 + CHUNK)
        pos = jnp.clip(idv - c0, 0, CHUNK - 1)
        for i in range(4):
            keep = in_r & (iota == i)
            g_t = plsc.load_gather(tbufs[i], [pos])
            g_d = plsc.load_gather(dbufs[i], [pos])
            tp_keep = jnp.where(keep, g_t, tp_keep)
            dp_keep = jnp.where(keep, g_d, dp_keep)

    rec = zero_i
    for i in range(4):
        mv = jnp.max(m[i])
        cand = jnp.where(m[i] == mv, mi[i], I32_MAX)
        best = jnp.min(cand)
        rec = jnp.where(iota == i, best, rec)

    safe_dp = jnp.where(dp_keep > 0, dp_keep, 1.0)
    acc = (dp_keep > 0) & (tp_keep / safe_dp >= uv)
    tok = jnp.where(acc, idv, rec)
    rej = jnp.where(acc | (iota >= 4), 0, 1).astype(jnp.int32)
    cs = plsc.cumsum(rej)
    excl = cs - rej
    write = (excl == 0) & (iota < 4)
    o = jnp.where(write, tok, jnp.int32(PLACEHOLDER))
    tot = jnp.max(jnp.where(iota < 4, cs, 0))
    bonusv = plsc.load_gather(bonus_v, [jnp.full((LANES,), w, jnp.int32)])
    o = jnp.where(iota == 4,
                  jnp.where(tot == 0, bonusv, jnp.int32(PLACEHOLDER)), o)
    orow_v[...] = o
    pltpu.sync_copy(orow_v, out_hbm.at[pl.ds(w * LANES, LANES)])


@jax.jit
def _sc_call(ids, dflat, tflat, bonus, u, qflat):
    mesh = plsc.VectorSubcoreMesh(core_axis_name="c", subcore_axis_name="s")
    f = pl.kernel(
        _sc_body,
        out_type=jax.ShapeDtypeStruct((B * LANES,), jnp.int32),
        mesh=mesh,
        scratch_types=[
            pltpu.VMEM((N,), jnp.int32),
            pltpu.VMEM((N,), jnp.float32),
            pltpu.VMEM((B,), jnp.int32),
            pltpu.VMEM((LANES,), jnp.int32),
        ] + [pltpu.VMEM((CHUNK,), jnp.float32)] * 18
          + [pltpu.SemaphoreType.DMA, pltpu.SemaphoreType.DMA],
        compiler_params=pltpu.CompilerParams(needs_layout_passes=False),
    )
    return f(ids, dflat, tflat, bonus, u, qflat)


def kernel(draft_token_ids, cu_num_draft_tokens, draft_probs, target_probs,
           bonus_token_ids, uniform_probs, q):
    del cu_num_draft_tokens
    out = _sc_call(
        draft_token_ids,
        draft_probs.reshape(-1),
        target_probs.reshape(-1),
        bonus_token_ids,
        uniform_probs,
        q.reshape(-1),
    )
    return out.reshape(B, LANES)[:, :L + 1]


# speedup vs baseline: 2.3133x; 1.5335x over previous
import jax
import jax.numpy as jnp
from jax import lax
from jax.experimental import pallas as pl
from jax.experimental.pallas import tpu as pltpu
from jax.experimental.pallas import tpu_sc as plsc

B = 32
L = 4
V = 100000
N = B * L
PLACEHOLDER = -1

LANES = 16
CH = 1536
NFULL = 64
SLOTS = NFULL // 2
TAIL0 = 98304
TAIL0_W = 1024
TAIL1 = 99328
TAIL1_W = 672
I32_MAX = 2147483647


def _sc_body(ids_hbm, d_hbm, t_hbm, bonus_hbm, u_hbm, q_hbm, out_hbm,
             ids_v, u_v, bonus_v, orow_v,
             tA, dA, qA, tB, dB, qB,
             tT0, dT0, qT0, tT1, dT1, qT1,
             mst, ist, dpst, tpst,
             pm, pi, pdp, ptp,
             stage_m, stage_i, stage_dp, stage_tp,
             semA, semB):
    c_ax = lax.axis_index("c")
    s_ax = lax.axis_index("s")
    w = c_ax * 16 + s_ax
    v = w // 2
    half = w % 2
    qtile = 8 * (v // 4)
    qs0 = 2 * (v % 4)
    iota = lax.iota(jnp.int32, LANES)

    tsets = (tA, tB)
    dsets = (dA, dB)
    qsets = (qA, qB)
    sems = (semA, semB)

    def issue(k, s):
        c0 = (2 * k + half) * CH
        return [
            pltpu.async_copy(t_hbm.at[pl.ds(8 * v, 8), pl.ds(c0, CH)],
                             tsets[s], sems[s]),
            pltpu.async_copy(d_hbm.at[pl.ds(8 * v, 8), pl.ds(c0, CH)],
                             dsets[s], sems[s]),
            pltpu.async_copy(q_hbm.at[pl.ds(qtile, 8), pl.ds(c0, CH)],
                             qsets[s], sems[s]),
        ]

    pltpu.sync_copy(ids_hbm, ids_v)
    pltpu.sync_copy(u_hbm, u_v)
    pltpu.sync_copy(bonus_hbm, bonus_v)

    idv8 = plsc.load_gather(ids_v, [jnp.minimum(8 * v + iota, jnp.int32(N - 1))])
    lane_row = jnp.minimum(iota, jnp.int32(7))

    zero_f = jnp.zeros((LANES,), jnp.float32)
    neg1_f = jnp.full((LANES,), -1.0, jnp.float32)
    zero_i = jnp.zeros((LANES,), jnp.int32)

    m = [neg1_f] * 8
    mi = [zero_i] * 8
    dp_keep = zero_f
    tp_keep = zero_f

    def make_inner(tbuf, dbuf, qbuf, c0):
        def inner(j, carry):
            ms = list(carry[:8])
            ix = list(carry[8:])
            sl = pl.ds(j * LANES, LANES)
            rq0 = 1.0 / qbuf[qs0, sl]
            rq1 = 1.0 / qbuf[qs0 + 1, sl]
            vidx = (jnp.int32(c0) + j * LANES) + iota
            for i in range(8):
                rq = rq0 if i < 4 else rq1
                p = jnp.maximum(tbuf[i, sl] - dbuf[i, sl], 0.0)
                r = p * rq
                better = r > ms[i]
                ms[i] = jnp.maximum(ms[i], r)
                ix[i] = jnp.where(better, vidx, ix[i])
            return tuple(ms) + tuple(ix)
        return inner

    def extract(tbuf, dbuf, c0, width, dp_keep, tp_keep):
        in_r = (idv8 >= c0) & (idv8 < c0 + width) & (iota < 8)
        pos = jnp.clip(idv8 - c0, 0, width - 1)
        g_t = plsc.load_gather(tbuf, [lane_row, pos])
        g_d = plsc.load_gather(dbuf, [lane_row, pos])
        return (jnp.where(in_r, g_d, dp_keep), jnp.where(in_r, g_t, tp_keep))

    cur = issue(0, 0)
    for k in range(SLOTS):
        s = k % 2
        c0 = (2 * k + half) * CH
        for cp in cur:
            cp.wait()
        if k + 1 < SLOTS:
            cur = issue(k + 1, (k + 1) % 2)
        carry = lax.fori_loop(0, CH // LANES,
                              make_inner(tsets[s], dsets[s], qsets[s], c0),
                              tuple(m) + tuple(mi), unroll=2)
        m = list(carry[:8])
        mi = list(carry[8:])
        dp_keep, tp_keep = extract(tsets[s], dsets[s], c0, CH, dp_keep, tp_keep)

    for i in range(8):
        mst[pl.ds(16 * i, 16)] = m[i]
        ist[pl.ds(16 * i, 16)] = mi[i]
    dpst[...] = dp_keep
    tpst[...] = tp_keep

    def tail(c0, width, tT, dT, qT):
        pltpu.sync_copy(t_hbm.at[pl.ds(8 * v, 8), pl.ds(c0, width)], tT)
        pltpu.sync_copy(d_hbm.at[pl.ds(8 * v, 8), pl.ds(c0, width)], dT)
        pltpu.sync_copy(q_hbm.at[pl.ds(qtile, 8), pl.ds(c0, width)], qT)
        m_l = [mst[pl.ds(16 * i, 16)] for i in range(8)]
        i_l = [ist[pl.ds(16 * i, 16)] for i in range(8)]
        carry = lax.fori_loop(0, width // LANES,
                              make_inner(tT, dT, qT, c0),
                              tuple(m_l) + tuple(i_l), unroll=2)
        for i in range(8):
            mst[pl.ds(16 * i, 16)] = carry[i]
            ist[pl.ds(16 * i, 16)] = carry[8 + i]
        ndp, ntp = extract(tT, dT, c0, width, dpst[...], tpst[...])
        dpst[...] = ndp
        tpst[...] = ntp

    @pl.when(half == 0)
    def _():
        tail(TAIL0, TAIL0_W, tT0, dT0, qT0)

    @pl.when(half == 1)
    def _():
        tail(TAIL1, TAIL1_W, tT1, dT1, qT1)

    pltpu.sync_copy(mst, stage_m.at[pl.ds(s_ax * 128, 128)])
    pltpu.sync_copy(ist, stage_i.at[pl.ds(s_ax * 128, 128)])
    pltpu.sync_copy(dpst, stage_dp.at[pl.ds(s_ax * 16, 16)])
    pltpu.sync_copy(tpst, stage_tp.at[pl.ds(s_ax * 16, 16)])
    plsc.subcore_barrier()
    part = s_ax ^ 1
    pltpu.sync_copy(stage_m.at[pl.ds(part * 128, 128)], pm)
    pltpu.sync_copy(stage_i.at[pl.ds(part * 128, 128)], pi)
    pltpu.sync_copy(stage_dp.at[pl.ds(part * 16, 16)], pdp)
    pltpu.sync_copy(stage_tp.at[pl.ds(part * 16, 16)], ptp)

    rec = zero_i
    for k in range(4):
        ti = 4 * half + k
        mA = mst[pl.ds(16 * ti, 16)]
        iA = ist[pl.ds(16 * ti, 16)]
        mB = pm[pl.ds(16 * ti, 16)]
        iB = pi[pl.ds(16 * ti, 16)]
        mv = jnp.maximum(jnp.max(mA), jnp.max(mB))
        bA = jnp.min(jnp.where(mA == mv, iA, I32_MAX))
        bB = jnp.min(jnp.where(mB == mv, iB, I32_MAX))
        rec = jnp.where(iota == k, jnp.minimum(bA, bB), rec)

    dpst[...] = dpst[...] + pdp[...]
    tpst[...] = tpst[...] + ptp[...]
    rowsel = jnp.minimum(4 * half + iota, jnp.int32(7))
    dpv = plsc.load_gather(dpst, [rowsel])
    tpv = plsc.load_gather(tpst, [rowsel])

    row_sel = jnp.minimum(4 * w + iota, jnp.int32(N - 1))
    idv = plsc.load_gather(ids_v, [row_sel])
    uv = plsc.load_gather(u_v, [row_sel])

    safe_dp = jnp.where(dpv > 0, dpv, 1.0)
    acc = (dpv > 0) & (tpv / safe_dp >= uv)
    tok = jnp.where(acc, idv, rec)
    rej = jnp.where(acc | (iota >= 4), 0, 1).astype(jnp.int32)
    cs = plsc.cumsum(rej)
    excl = cs - rej
    write = (excl == 0) & (iota < 4)
    o = jnp.where(write, tok, jnp.int32(PLACEHOLDER))
    tot = jnp.max(jnp.where(iota < 4, cs, 0))
    bonusv = plsc.load_gather(bonus_v, [jnp.full((LANES,), w, jnp.int32)])
    o = jnp.where(iota == 4,
                  jnp.where(tot == 0, bonusv, jnp.int32(PLACEHOLDER)), o)
    orow_v[...] = o
    pltpu.sync_copy(orow_v, out_hbm.at[pl.ds(w * LANES, LANES)])


@jax.jit
def _sc_call(ids, d2, t2, bonus, u, q2):
    mesh = plsc.VectorSubcoreMesh(core_axis_name="c", subcore_axis_name="s")
    f = pl.kernel(
        _sc_body,
        out_type=jax.ShapeDtypeStruct((B * LANES,), jnp.int32),
        mesh=mesh,
        scratch_types=[
            pltpu.VMEM((N,), jnp.int32),
            pltpu.VMEM((N,), jnp.float32),
            pltpu.VMEM((B,), jnp.int32),
            pltpu.VMEM((LANES,), jnp.int32),
        ] + [pltpu.VMEM((8, CH), jnp.float32)] * 6
          + [pltpu.VMEM((8, TAIL0_W), jnp.float32)] * 3
          + [pltpu.VMEM((8, TAIL1_W), jnp.float32)] * 3 + [
            pltpu.VMEM((128,), jnp.float32),
            pltpu.VMEM((128,), jnp.int32),
            pltpu.VMEM((LANES,), jnp.float32),
            pltpu.VMEM((LANES,), jnp.float32),
            pltpu.VMEM((128,), jnp.float32),
            pltpu.VMEM((128,), jnp.int32),
            pltpu.VMEM((LANES,), jnp.float32),
            pltpu.VMEM((LANES,), jnp.float32),
            pltpu.VMEM_SHARED((16 * 128,), jnp.float32),
            pltpu.VMEM_SHARED((16 * 128,), jnp.int32),
            pltpu.VMEM_SHARED((16 * 16,), jnp.float32),
            pltpu.VMEM_SHARED((16 * 16,), jnp.float32),
            pltpu.SemaphoreType.DMA, pltpu.SemaphoreType.DMA,
        ],
        compiler_params=pltpu.CompilerParams(
            needs_layout_passes=False, use_tc_tiling_on_sc=True),
    )
    return f(ids, d2, t2, bonus, u, q2)


def kernel(draft_token_ids, cu_num_draft_tokens, draft_probs, target_probs,
           bonus_token_ids, uniform_probs, q):
    del cu_num_draft_tokens
    out = _sc_call(draft_token_ids, draft_probs, target_probs,
                   bonus_token_ids, uniform_probs, q)
    return out.reshape(B, LANES)[:, :L + 1]
